# baseline (speedup 1.0000x reference)
import functools

import jax
import jax.numpy as jnp
from jax import lax
from jax.experimental import pallas as pl
from jax.experimental.pallas import tpu as pltpu

_N = 5000
_NCLS = 21
_NP = 5120
_T = 256
_NT = _NP // _T
_NMS_T = 0.3
_SCORE_T = 0.05
_IMG_H = 600.0
_IMG_W = 800.0


def _nms_body(rois_w_ref, loc_w_ref, scores_ref, ord_r_ref,
              ord_t_ref, out_ref, oh_s, ohT_s, cols_s, keeps_s, kf_s, a_s,
              decw_s):
    f32 = jnp.float32
    l = pl.program_id(0)

    z = scores_ref[...]
    zmax = jnp.max(z, axis=0, keepdims=True)
    ez = jnp.exp(z - zmax)
    den = jnp.sum(ez, axis=0, keepdims=True)
    sel = (lax.broadcasted_iota(jnp.int32, (24, 1), 0) == l + 1).astype(f32)
    prob_l = jnp.sum((ez / den) * sel, axis=0, keepdims=True)

    y1 = rois_w_ref[0:1, :]
    x1 = rois_w_ref[1:2, :]
    y2 = rois_w_ref[2:3, :]
    x2 = rois_w_ref[3:4, :]
    sh = y2 - y1
    sw = x2 - x1
    scy = y1 + 0.5 * sh
    scx = x1 + 0.5 * sw
    lw = loc_w_ref[0]
    dy = lw[0:1, :] * 0.1
    dx = lw[1:2, :] * 0.1
    dh = lw[2:3, :] * 0.2
    dw = lw[3:4, :] * 0.2 + 0.2
    cy = dy * sh + scy
    cx = dx * sw + scx
    hh = jnp.exp(dh) * sh
    ww = jnp.exp(dw) * sw
    yy1 = jnp.clip(cy - 0.5 * hh, 0.0, _IMG_H)
    xx1 = jnp.clip(cx - 0.5 * ww, 0.0, _IMG_W)
    yy2 = jnp.clip(cy + 0.5 * hh, 0.0, _IMG_H)
    xx2 = jnp.clip(cx + 0.5 * ww, 0.0, _IMG_W)
    dec = jnp.concatenate([yy1, xx1, yy2, xx2,
                           jnp.zeros((4, _NP), f32)], axis=0)
    p0 = dec.astype(jnp.bfloat16).astype(f32)
    r0 = dec - p0
    p1 = r0.astype(jnp.bfloat16).astype(f32)
    p2 = r0 - p1
    decw_s[0:8, :] = p0
    decw_s[8:16, :] = p1
    decw_s[16:24, :] = p2

    mask_row = (prob_l > _SCORE_T).astype(f32)
    m_cnt = jnp.sum(mask_row).astype(jnp.int32)
    nt = (m_cnt + _T - 1) // _T

    kf_s[...] = jnp.zeros((8, _NP), f32)

    iota_col_t = lax.broadcasted_iota(jnp.int32, (_T, 1), 0)
    iota_row_t = lax.broadcasted_iota(jnp.int32, (1, _T), 1)
    iota_col_np = lax.broadcasted_iota(jnp.int32, (_NP, 1), 0)
    iota_row_np = lax.broadcasted_iota(jnp.int32, (1, _NP), 1)

    def tile_body(t, carry):
        base = t * _T
        idx_r = ord_r_ref[0, 0:1, pl.ds(base, _T)]
        idx_c = ord_t_ref[0, pl.ds(base, _T), 0:1]
        ohT_s[...] = (iota_col_np == idx_r).astype(f32)
        oh_s[...] = (idx_c == iota_row_np).astype(f32)
        oht = ohT_s[...]
        grow = (jnp.dot(decw_s[0:8, :], oht, preferred_element_type=f32)
                + jnp.dot(decw_s[8:16, :], oht, preferred_element_type=f32)
                + jnp.dot(decw_s[16:24, :], oht,
                          preferred_element_type=f32))
        gcol = jnp.transpose(grow, (1, 0))
        y1r = grow[0:1, :]
        x1r = grow[1:2, :]
        y2r = grow[2:3, :]
        x2r = grow[3:4, :]
        area_r = jnp.maximum(y2r - y1r, 0.0) * jnp.maximum(x2r - x1r, 0.0)
        y1c = gcol[:, 0:1]
        x1c = gcol[:, 1:2]
        y2c = gcol[:, 2:3]
        x2c = gcol[:, 3:4]
        area_c = jnp.maximum(y2c - y1c, 0.0) * jnp.maximum(x2c - x1c, 0.0)
        cols_s[t] = gcol

        def cross(s, sup):
            cs = cols_s[s]
            ks = keeps_s[s]
            sy1 = cs[:, 0:1]
            sx1 = cs[:, 1:2]
            sy2 = cs[:, 2:3]
            sx2 = cs[:, 3:4]
            s_area = jnp.maximum(sy2 - sy1, 0.0) * jnp.maximum(sx2 - sx1, 0.0)
            tly = jnp.maximum(sy1, y1r)
            tlx = jnp.maximum(sx1, x1r)
            bry = jnp.minimum(sy2, y2r)
            brx = jnp.minimum(sx2, x2r)
            iw = jnp.clip(brx - tlx, 0.0, None)
            ih = jnp.clip(bry - tly, 0.0, None)
            inter = iw * ih
            iou = inter / (s_area + area_r - inter + 1e-8)
            af = (iou > _NMS_T).astype(f32)
            hits = jnp.dot(ks, af, preferred_element_type=f32)[0:1, :]
            return jnp.maximum(sup, jnp.minimum(hits, 1.0))

        sup_x = lax.fori_loop(0, t, cross, jnp.zeros((1, _T), f32))

        tly = jnp.maximum(y1c, y1r)
        tlx = jnp.maximum(x1c, x1r)
        bry = jnp.minimum(y2c, y2r)
        brx = jnp.minimum(x2c, x2r)
        iw = jnp.clip(brx - tlx, 0.0, None)
        ih = jnp.clip(bry - tly, 0.0, None)
        inter = iw * ih
        iou_tt = inter / (area_c + area_r - inter + 1e-8)
        a_s[...] = jnp.where((iou_tt > _NMS_T) & (iota_col_t < iota_row_t),
                             1.0, 0.0)

        v_row = ((base + iota_row_t) < m_cnt).astype(f32)
        base_k = jnp.where(sup_x > 0.0, 0.0, v_row)

        def jcond(c):
            k, p, it = c
            return jnp.logical_and(it < _T + 2, jnp.any(k != p))

        def jbody(c):
            k, p, it = c
            k8 = jnp.broadcast_to(k, (8, _T))
            hits = jnp.dot(k8, a_s[...], preferred_element_type=f32)[0:1, :]
            nk = jnp.where(hits > 0.0, 0.0, base_k)
            return (nk, k, it + 1)

        keep, _, _ = lax.while_loop(
            jcond, jbody, (base_k, base_k - 1.0, jnp.int32(0)))

        keep8 = jnp.broadcast_to(keep, (8, _T))
        keeps_s[t] = keep8
        kf_s[...] = kf_s[...] + jnp.dot(keep8, oh_s[...],
                                        preferred_element_type=f32)
        return carry

    lax.fori_loop(0, nt, tile_body, jnp.int32(0))

    kf = kf_s[0:1, :] * mask_row
    out_ref[0, 0:1, :] = yy1 * kf
    out_ref[0, 1:2, :] = xx1 * kf
    out_ref[0, 2:3, :] = yy2 * kf
    out_ref[0, 3:4, :] = xx2 * kf
    out_ref[0, 4:5, :] = prob_l * kf
    out_ref[0, 5:8, :] = jnp.zeros((3, _NP), f32)


@jax.jit
def kernel(rois, roi_cls_loc, roi_scores):
    f32 = jnp.float32
    n = rois.shape[0]
    pad = _NP - n
    ncf = _NCLS - 1

    rois_w = jnp.pad(rois.astype(f32).T, ((0, 4), (0, pad)))
    loc = roi_cls_loc.astype(f32).reshape(n, _NCLS, 4)
    loc_w = jnp.pad(jnp.transpose(loc, (1, 2, 0)),
                    ((0, 0), (0, 4), (0, pad)))[1:]
    scores_w = jnp.pad(roi_scores.astype(f32).T, ((0, 3), (0, pad)),
                       constant_values=-1e30)

    prob = jax.nn.softmax(roi_scores.astype(f32), axis=1)
    s = jnp.where(prob > _SCORE_T, prob, -jnp.inf)[:, 1:]
    s = jnp.pad(s, ((0, pad), (0, 0)), constant_values=-jnp.inf)
    order = jnp.argsort(-s.T, axis=1).astype(jnp.int32)
    ord_r = order.reshape(ncf, 1, _NP)
    ord_t = order.reshape(ncf, _NP, 1)

    out = pl.pallas_call(
        _nms_body,
        grid=(ncf,),
        in_specs=[
            pl.BlockSpec((8, _NP), lambda l: (0, 0)),
            pl.BlockSpec((1, 8, _NP), lambda l: (l, 0, 0)),
            pl.BlockSpec((24, _NP), lambda l: (0, 0)),
            pl.BlockSpec((1, 1, _NP), lambda l: (l, 0, 0)),
            pl.BlockSpec((1, _NP, 1), lambda l: (l, 0, 0)),
        ],
        out_specs=pl.BlockSpec((1, 8, _NP), lambda l: (l, 0, 0)),
        out_shape=jax.ShapeDtypeStruct((ncf, 8, _NP), f32),
        scratch_shapes=[
            pltpu.VMEM((_T, _NP), f32),
            pltpu.VMEM((_NP, _T), f32),
            pltpu.VMEM((_NT, _T, 8), f32),
            pltpu.VMEM((_NT, 8, _T), f32),
            pltpu.VMEM((8, _NP), f32),
            pltpu.VMEM((_T, _T), f32),
            pltpu.VMEM((24, _NP), f32),
        ],
    )(rois_w, loc_w, scores_w, ord_r, ord_t)

    return out[:, :5, :n].transpose(0, 2, 1)


# speedup vs baseline: 1.2660x; 1.2660x over previous
import jax
import jax.numpy as jnp
from jax import lax
from jax.experimental import pallas as pl
from jax.experimental.pallas import tpu as pltpu

_N = 5000
_NCLS = 21
_NP = 5120
_NB = 40
_LB = 128
_T = 256
_NT = _NP // _T
_NMS_T = 0.3
_SCORE_T = 0.05
_IMG_H = 600.0
_IMG_W = 800.0


def _nms_body(rois_ref, loc_ref, scores_ref, ord_r_ref, ord_t_ref, out_ref,
              cols_s, keeps_s, kf_s, a_s, dec_s):
    f32 = jnp.float32
    l = pl.program_id(0)

    z = scores_ref[...]
    zmax = jnp.max(z, axis=0)
    ez = jnp.exp(z - zmax[None])
    den = jnp.sum(ez, axis=0)
    sel = (lax.broadcasted_iota(jnp.int32, (24, 1, 1), 0) == l + 1).astype(f32)
    prob_l = jnp.sum(ez * sel, axis=0) / den

    y1 = rois_ref[0]
    x1 = rois_ref[1]
    y2 = rois_ref[2]
    x2 = rois_ref[3]
    sh = y2 - y1
    sw = x2 - x1
    scy = y1 + 0.5 * sh
    scx = x1 + 0.5 * sw
    lb = loc_ref[0]
    dy = lb[0] * 0.1
    dx = lb[1] * 0.1
    dh = lb[2] * 0.2
    dw = lb[3] * 0.2 + 0.2
    cy = dy * sh + scy
    cx = dx * sw + scx
    hh = jnp.exp(dh) * sh
    ww = jnp.exp(dw) * sw
    yy1 = jnp.clip(cy - 0.5 * hh, 0.0, _IMG_H)
    xx1 = jnp.clip(cx - 0.5 * ww, 0.0, _IMG_W)
    yy2 = jnp.clip(cy + 0.5 * hh, 0.0, _IMG_H)
    xx2 = jnp.clip(cx + 0.5 * ww, 0.0, _IMG_W)
    dec = jnp.concatenate([yy1, xx1, yy2, xx2], axis=1)
    p0 = dec.astype(jnp.bfloat16).astype(f32)
    r0 = dec - p0
    p1 = r0.astype(jnp.bfloat16).astype(f32)
    p2 = r0 - p1
    dec_s[0:_NB, :] = p0
    dec_s[_NB:2 * _NB, :] = p1
    dec_s[2 * _NB:3 * _NB, :] = p2

    mask = (prob_l > _SCORE_T).astype(f32)
    m_cnt = jnp.sum(mask).astype(jnp.int32)
    nt = (m_cnt + _T - 1) // _T

    kf_s[...] = jnp.zeros((_NB, _LB), f32)

    iota_col_t = lax.broadcasted_iota(jnp.int32, (_T, 1), 0)
    iota_row_t = lax.broadcasted_iota(jnp.int32, (1, _T), 1)
    iota_row_nb = lax.broadcasted_iota(jnp.int32, (1, _NB), 1)
    iota_col_nb = lax.broadcasted_iota(jnp.int32, (_NB, 1), 0)
    iota_row_lb = lax.broadcasted_iota(jnp.int32, (1, _LB), 1)

    def tile_body(t, carry):
        base = t * _T
        idx_r = ord_r_ref[0, 0:1, pl.ds(base, _T)]
        idx_c = ord_t_ref[0, pl.ds(base, _T), 0:1]
        hi_r = idx_r >> 7
        hi_c = idx_c >> 7
        lo_c = idx_c & 127
        oh_hi = (hi_c == iota_row_nb).astype(f32)
        oh_hi_t = (iota_col_nb == hi_r).astype(f32)
        m_lo = (lo_c == iota_row_lb).astype(f32)
        y = (jnp.dot(oh_hi, dec_s[0:_NB, :], preferred_element_type=f32)
             + jnp.dot(oh_hi, dec_s[_NB:2 * _NB, :], preferred_element_type=f32)
             + jnp.dot(oh_hi, dec_s[2 * _NB:3 * _NB, :],
                       preferred_element_type=f32))
        gc = [jnp.sum(y[:, c * _LB:(c + 1) * _LB] * m_lo, axis=1,
                      keepdims=True) for c in range(4)]
        gcol = jnp.concatenate(gc + [jnp.zeros((_T, 4), f32)], axis=1)
        grow = jnp.transpose(gcol, (1, 0))
        y1r = grow[0:1, :]
        x1r = grow[1:2, :]
        y2r = grow[2:3, :]
        x2r = grow[3:4, :]
        area_r = jnp.maximum(y2r - y1r, 0.0) * jnp.maximum(x2r - x1r, 0.0)
        y1c = gcol[:, 0:1]
        x1c = gcol[:, 1:2]
        y2c = gcol[:, 2:3]
        x2c = gcol[:, 3:4]
        area_c = jnp.maximum(y2c - y1c, 0.0) * jnp.maximum(x2c - x1c, 0.0)
        cols_s[t] = gcol

        def cross(s, sup):
            cs = cols_s[s]
            ks = keeps_s[s]
            sy1 = cs[:, 0:1]
            sx1 = cs[:, 1:2]
            sy2 = cs[:, 2:3]
            sx2 = cs[:, 3:4]
            s_area = jnp.maximum(sy2 - sy1, 0.0) * jnp.maximum(sx2 - sx1, 0.0)
            tly = jnp.maximum(sy1, y1r)
            tlx = jnp.maximum(sx1, x1r)
            bry = jnp.minimum(sy2, y2r)
            brx = jnp.minimum(sx2, x2r)
            iw = jnp.clip(brx - tlx, 0.0, None)
            ih = jnp.clip(bry - tly, 0.0, None)
            inter = iw * ih
            iou = inter / (s_area + area_r - inter + 1e-8)
            af = (iou > _NMS_T).astype(f32)
            hits = jnp.dot(ks, af, preferred_element_type=f32)[0:1, :]
            return jnp.maximum(sup, jnp.minimum(hits, 1.0))

        sup_x = lax.fori_loop(0, t, cross, jnp.zeros((1, _T), f32))

        tly = jnp.maximum(y1c, y1r)
        tlx = jnp.maximum(x1c, x1r)
        bry = jnp.minimum(y2c, y2r)
        brx = jnp.minimum(x2c, x2r)
        iw = jnp.clip(brx - tlx, 0.0, None)
        ih = jnp.clip(bry - tly, 0.0, None)
        inter = iw * ih
        iou_tt = inter / (area_c + area_r - inter + 1e-8)
        a_s[...] = jnp.where((iou_tt > _NMS_T) & (iota_col_t < iota_row_t),
                             1.0, 0.0)

        v_row = ((base + iota_row_t) < m_cnt).astype(f32)
        base_k = jnp.where(sup_x > 0.0, 0.0, v_row)

        def jcond(c):
            k, p, it = c
            return jnp.logical_and(it < _T + 2, jnp.any(k != p))

        def jbody(c):
            k, p, it = c
            k8 = jnp.broadcast_to(k, (8, _T))
            hits = jnp.dot(k8, a_s[...], preferred_element_type=f32)[0:1, :]
            nk = jnp.where(hits > 0.0, 0.0, base_k)
            return (nk, k, it + 1)

        keep, _, _ = lax.while_loop(
            jcond, jbody, (base_k, base_k - 1.0, jnp.int32(0)))

        keeps_s[t] = jnp.broadcast_to(keep, (8, _T))
        kf_s[...] = kf_s[...] + jnp.dot(oh_hi_t * keep, m_lo,
                                        preferred_element_type=f32)
        return carry

    lax.fori_loop(0, nt, tile_body, jnp.int32(0))

    kf = kf_s[...] * mask
    out_ref[0, 0] = yy1 * kf
    out_ref[0, 1] = xx1 * kf
    out_ref[0, 2] = yy2 * kf
    out_ref[0, 3] = xx2 * kf
    out_ref[0, 4] = prob_l * kf
    out_ref[0, 5] = jnp.zeros((_NB, _LB), f32)
    out_ref[0, 6] = jnp.zeros((_NB, _LB), f32)
    out_ref[0, 7] = jnp.zeros((_NB, _LB), f32)


@jax.jit
def kernel(rois, roi_cls_loc, roi_scores):
    f32 = jnp.float32
    n = rois.shape[0]
    pad = _NP - n
    ncf = _NCLS - 1

    rois_b = jnp.pad(rois.astype(f32).T, ((0, 0), (0, pad))).reshape(
        4, _NB, _LB)
    loc = roi_cls_loc.astype(f32).reshape(n, _NCLS, 4)
    loc_b = jnp.pad(jnp.transpose(loc, (1, 2, 0)),
                    ((0, 0), (0, 0), (0, pad)))[1:].reshape(ncf, 4, _NB, _LB)
    scores_b = jnp.pad(roi_scores.astype(f32).T, ((0, 3), (0, pad)),
                       constant_values=-1e30).reshape(24, _NB, _LB)

    prob = jax.nn.softmax(roi_scores.astype(f32), axis=1)
    s = jnp.where(prob > _SCORE_T, prob, -jnp.inf)[:, 1:]
    s = jnp.pad(s, ((0, pad), (0, 0)), constant_values=-jnp.inf)
    order = jnp.argsort(-s.T, axis=1).astype(jnp.int32)
    ord_r = order.reshape(ncf, 1, _NP)
    ord_t = order.reshape(ncf, _NP, 1)

    out = pl.pallas_call(
        _nms_body,
        grid=(ncf,),
        in_specs=[
            pl.BlockSpec((4, _NB, _LB), lambda l: (0, 0, 0)),
            pl.BlockSpec((1, 4, _NB, _LB), lambda l: (l, 0, 0, 0)),
            pl.BlockSpec((24, _NB, _LB), lambda l: (0, 0, 0)),
            pl.BlockSpec((1, 1, _NP), lambda l: (l, 0, 0)),
            pl.BlockSpec((1, _NP, 1), lambda l: (l, 0, 0)),
        ],
        out_specs=pl.BlockSpec((1, 8, _NB, _LB), lambda l: (l, 0, 0, 0)),
        out_shape=jax.ShapeDtypeStruct((ncf, 8, _NB, _LB), f32),
        scratch_shapes=[
            pltpu.VMEM((_NT, _T, 8), f32),
            pltpu.VMEM((_NT, 8, _T), f32),
            pltpu.VMEM((_NB, _LB), f32),
            pltpu.VMEM((_T, _T), f32),
            pltpu.VMEM((3 * _NB, 4 * _LB), f32),
        ],
    )(rois_b, loc_b, scores_b, ord_r, ord_t)

    return out.reshape(ncf, 8, _NP)[:, :5, :n].transpose(0, 2, 1)
